# Initial kernel scaffold; baseline (speedup 1.0000x reference)
import jax
import jax.numpy as jnp
from jax.experimental import pallas as pl


def kernel(hidden_states, gate_w, e_score_correction_bias, w_gate_proj, w_up_proj, w_down_proj, ws_gate_proj, ws_up_proj, ws_down_proj):
    raise NotImplementedError("write your pallas kernel here")


# speedup vs baseline: 1.7018x; 1.7018x over previous
import functools

import jax
import jax.numpy as jnp
from jax import lax
from jax.experimental import pallas as pl
from jax.experimental.pallas import tpu as pltpu
from jax.experimental.pallas import tpu_sc as plsc

T = 2048
D = 1024
E = 16
K = 2
F = 512
NG = 4
SH_F = 1024
RSF = 2.5

NC = 2
NS = 16
NW = NC * NS
TPW = T // NW
B = 128
NB = (T * K) // B + E
NPAD = NB * B
NEG = -1e30


def _router_body(x_ref, gw_ref, b_ref, i1_ref, i2_ref, w1_ref, w2_ref,
                 cnt_ref):
    xb = x_ref[...]
    gw = gw_ref[...]
    logits = jnp.dot(xb, gw, preferred_element_type=jnp.float32)
    s = jax.nn.sigmoid(logits)
    sc = s + b_ref[...]

    gs = []
    for g in range(NG):
        a = sc[:, 4 * g + 0:4 * g + 1]
        b2 = sc[:, 4 * g + 1:4 * g + 2]
        c = sc[:, 4 * g + 2:4 * g + 3]
        d = sc[:, 4 * g + 3:4 * g + 4]
        hi1 = jnp.maximum(a, b2)
        lo1 = jnp.minimum(a, b2)
        hi2 = jnp.maximum(c, d)
        lo2 = jnp.minimum(c, d)
        top1 = jnp.maximum(hi1, hi2)
        sec = jnp.maximum(jnp.minimum(hi1, hi2),
                          jnp.where(hi1 >= hi2, lo1, lo2))
        gs.append(top1 + sec)

    bv = gs[0]
    bi = jnp.zeros_like(gs[0], dtype=jnp.int32)
    for g in range(1, NG):
        take = gs[g] > bv
        bv = jnp.where(take, gs[g], bv)
        bi = jnp.where(take, g, bi)
    g1 = bi
    bv = jnp.where(g1 == 0, NEG, gs[0])
    bi = jnp.zeros_like(g1)
    for g in range(1, NG):
        cand = jnp.where(g1 == g, NEG, gs[g])
        take = cand > bv
        bv = jnp.where(take, cand, bv)
        bi = jnp.where(take, g, bi)
    g2 = bi

    tmp = []
    for e in range(E):
        grp = e // (E // NG)
        m = (g1 == grp) | (g2 == grp)
        tmp.append(jnp.where(m, sc[:, e:e + 1], 0.0))

    bv = tmp[0]
    bi = jnp.zeros_like(g1)
    bs = s[:, 0:1]
    for e in range(1, E):
        take = tmp[e] > bv
        bv = jnp.where(take, tmp[e], bv)
        bi = jnp.where(take, e, bi)
        bs = jnp.where(take, s[:, e:e + 1], bs)
    i1 = bi
    s1 = bs
    bv = jnp.where(i1 == 0, NEG, tmp[0])
    bi = jnp.zeros_like(g1)
    bs = s[:, 0:1]
    for e in range(1, E):
        cand = jnp.where(i1 == e, NEG, tmp[e])
        take = cand > bv
        bv = jnp.where(take, cand, bv)
        bi = jnp.where(take, e, bi)
        bs = jnp.where(take, s[:, e:e + 1], bs)
    i2 = bi
    s2 = bs

    tot = s1 + s2 + 1e-20
    i1_ref[...] = i1
    i2_ref[...] = i2
    w1_ref[...] = s1 / tot * RSF
    w2_ref[...] = s2 / tot * RSF

    cols = []
    for e in range(E):
        ce = jnp.sum((i1 == e).astype(jnp.int32) + (i2 == e).astype(jnp.int32))
        cols.append(jnp.reshape(ce, (1, 1)))
    cnt_ref[...] = jnp.reshape(jnp.concatenate(cols, axis=1), (1, 1, E))


def _router(x, gate_w, bias2d, interpret=False):
    out_shapes = (
        jax.ShapeDtypeStruct((T, 1), jnp.int32),
        jax.ShapeDtypeStruct((T, 1), jnp.int32),
        jax.ShapeDtypeStruct((T, 1), jnp.float32),
        jax.ShapeDtypeStruct((T, 1), jnp.float32),
        jax.ShapeDtypeStruct((NW, 1, E), jnp.int32),
    )
    return pl.pallas_call(
        _router_body,
        grid=(NW,),
        in_specs=[
            pl.BlockSpec((TPW, D), lambda i: (i, 0)),
            pl.BlockSpec((D, E), lambda i: (0, 0)),
            pl.BlockSpec((1, E), lambda i: (0, 0)),
        ],
        out_specs=(
            pl.BlockSpec((TPW, 1), lambda i: (i, 0)),
            pl.BlockSpec((TPW, 1), lambda i: (i, 0)),
            pl.BlockSpec((TPW, 1), lambda i: (i, 0)),
            pl.BlockSpec((TPW, 1), lambda i: (i, 0)),
            pl.BlockSpec((1, 1, E), lambda i: (i, 0, 0)),
        ),
        out_shape=out_shapes,
        interpret=interpret,
    )(x, gate_w, bias2d)


def _dispatch_body(x_hbm, i1_hbm, i2_hbm, cnt_hbm,
                   xg_hbm, p1_hbm, p2_hbm, be_hbm,
                   cntbuf, p1buf, p2buf, i1buf, i2buf,
                   xrows, bebuf, sem1, sem2):
    cid = lax.axis_index("c")
    sid = lax.axis_index("s")
    wid = sid * NC + cid
    t0 = wid * TPW

    pltpu.sync_copy(cnt_hbm, cntbuf)
    pltpu.sync_copy(i1_hbm.at[pl.ds(t0, TPW)], i1buf)
    pltpu.sync_copy(i2_hbm.at[pl.ds(t0, TPW)], i2buf)

    tot = jnp.zeros((E,), jnp.int32)
    pre = jnp.zeros((E,), jnp.int32)
    for w in range(NW):
        row = cntbuf[w, :]
        tot = tot + row
        pre = pre + row * (w < wid).astype(jnp.int32)
    padcnt = ((tot + (B - 1)) >> 7) << 7
    csum = jnp.cumsum(padcnt)
    off = csum - padcnt
    base = off + pre

    @pl.when(wid == 0)
    def _():
        nxt = csum >> 7
        for c in range(NB // 16):
            ids = lax.iota(jnp.int32, 16) + c * 16
            acc = jnp.zeros((16,), jnp.int32)
            for e in range(E):
                acc = acc + (ids >= nxt[e]).astype(jnp.int32)
            bebuf[pl.ds(c * 16, 16)] = jnp.minimum(acc, E - 1)
        pltpu.sync_copy(bebuf, be_hbm)

    lane_ids = lax.iota(jnp.int32, 16)
    for c in range(TPW // 16):
        for ibuf, pbuf in ((i1buf, p1buf), (i2buf, p2buf)):
            iv = ibuf[pl.ds(c * 16, 16)]
            pos = jnp.zeros((16,), jnp.int32)
            for e in range(E):
                m = iv == e
                mi = m.astype(jnp.int32)
                cs = jnp.cumsum(mi)
                pos = pos + jnp.where(m, base[e] + cs - 1, 0)
                base = base + jnp.where(lane_ids == e, jnp.sum(mi), 0)
            pbuf[c, :] = pos
        pltpu.sync_copy(x_hbm.at[pl.ds(t0 + c * 16, 16)], xrows)
        cp1 = pltpu.async_copy(xrows, xg_hbm.at[p1buf.at[c]], sem1)
        cp2 = pltpu.async_copy(xrows, xg_hbm.at[p2buf.at[c]], sem2)
        cp1.wait()
        cp2.wait()

    pltpu.sync_copy(p1buf, p1_hbm.at[pl.ds(wid * 4, 4)])
    pltpu.sync_copy(p2buf, p2_hbm.at[pl.ds(wid * 4, 4)])


def _dispatch(x, i1, i2, cnt, interpret=False):
    mesh = plsc.VectorSubcoreMesh(core_axis_name="c", subcore_axis_name="s")
    kfn = pl.kernel(
        _dispatch_body,
        out_type=(
            jax.ShapeDtypeStruct((NPAD, D), jnp.float32),
            jax.ShapeDtypeStruct((T // 16, 16), jnp.int32),
            jax.ShapeDtypeStruct((T // 16, 16), jnp.int32),
            jax.ShapeDtypeStruct((NB,), jnp.int32),
        ),
        mesh=mesh,
        scratch_types=[
            pltpu.VMEM((NW, E), jnp.int32),
            pltpu.VMEM((4, 16), jnp.int32),
            pltpu.VMEM((4, 16), jnp.int32),
            pltpu.VMEM((TPW,), jnp.int32),
            pltpu.VMEM((TPW,), jnp.int32),
            pltpu.VMEM((16, D), jnp.float32),
            pltpu.VMEM((NB,), jnp.int32),
            pltpu.SemaphoreType.DMA,
            pltpu.SemaphoreType.DMA,
        ],
        compiler_params=pltpu.CompilerParams(needs_layout_passes=False),
        interpret=interpret,
    )
    return kfn(x, i1, i2, cnt)


def _gemm_body(be_ref, xg_ref, wg_ref, wu_ref, wd_ref, y_ref):
    xb = xg_ref[...]
    h1 = jnp.dot(xb, wg_ref[0], preferred_element_type=jnp.float32)
    h2 = jnp.dot(xb, wu_ref[0], preferred_element_type=jnp.float32)
    act = h1 * jax.nn.sigmoid(h1) * h2
    y_ref[...] = jnp.dot(act, wd_ref[0], preferred_element_type=jnp.float32)


def _grouped_gemm(be, xg, wg, wu, wd, interpret=False):
    grid_spec = pltpu.PrefetchScalarGridSpec(
        num_scalar_prefetch=1,
        grid=(NB,),
        in_specs=[
            pl.BlockSpec((B, D), lambda i, be_r: (i, 0)),
            pl.BlockSpec((1, D, F), lambda i, be_r: (be_r[i], 0, 0)),
            pl.BlockSpec((1, D, F), lambda i, be_r: (be_r[i], 0, 0)),
            pl.BlockSpec((1, F, D), lambda i, be_r: (be_r[i], 0, 0)),
        ],
        out_specs=pl.BlockSpec((B, D), lambda i, be_r: (i, 0)),
    )
    return pl.pallas_call(
        _gemm_body,
        grid_spec=grid_spec,
        out_shape=jax.ShapeDtypeStruct((NPAD, D), jnp.float32),
        interpret=interpret,
    )(be, xg, wg, wu, wd)


def _shared_body(x_ref, wg_ref, wu_ref, wd_ref, o_ref):
    xb = x_ref[...]
    h1 = jnp.dot(xb, wg_ref[...], preferred_element_type=jnp.float32)
    h2 = jnp.dot(xb, wu_ref[...], preferred_element_type=jnp.float32)
    act = h1 * jax.nn.sigmoid(h1) * h2
    o_ref[...] = jnp.dot(act, wd_ref[...], preferred_element_type=jnp.float32)


def _shared(x, wsg, wsu, wsd, interpret=False):
    MB = 256
    return pl.pallas_call(
        _shared_body,
        grid=(T // MB,),
        in_specs=[
            pl.BlockSpec((MB, D), lambda i: (i, 0)),
            pl.BlockSpec((D, SH_F), lambda i: (0, 0)),
            pl.BlockSpec((D, SH_F), lambda i: (0, 0)),
            pl.BlockSpec((SH_F, D), lambda i: (0, 0)),
        ],
        out_specs=pl.BlockSpec((MB, D), lambda i: (i, 0)),
        out_shape=jax.ShapeDtypeStruct((T, D), jnp.float32),
        interpret=interpret,
    )(x, wsg, wsu, wsd)


def _combine_body(y_hbm, p1_hbm, p2_hbm, w1_hbm, w2_hbm, sh_hbm, out_hbm,
                  p1buf, p2buf, w1buf, w2buf, y1, y2, shb, ob,
                  sem1, sem2, sem3):
    cid = lax.axis_index("c")
    sid = lax.axis_index("s")
    wid = sid * NC + cid
    t0 = wid * TPW

    pltpu.sync_copy(p1_hbm.at[pl.ds(wid * 4, 4)], p1buf)
    pltpu.sync_copy(p2_hbm.at[pl.ds(wid * 4, 4)], p2buf)
    pltpu.sync_copy(w1_hbm.at[pl.ds(t0, TPW)], w1buf)
    pltpu.sync_copy(w2_hbm.at[pl.ds(t0, TPW)], w2buf)

    for c in range(TPW // 16):
        g1 = pltpu.async_copy(y_hbm.at[p1buf.at[c]], y1, sem1)
        g2 = pltpu.async_copy(y_hbm.at[p2buf.at[c]], y2, sem2)
        gs = pltpu.async_copy(sh_hbm.at[pl.ds(t0 + c * 16, 16)], shb, sem3)
        g1.wait()
        g2.wait()
        gs.wait()
        w1v = w1buf[pl.ds(c * 16, 16)]
        w2v = w2buf[pl.ds(c * 16, 16)]
        for j in range(16):
            a1 = w1v[j]
            a2 = w2v[j]

            def body(v, _, j=j, a1=a1, a2=a2):
                sl = pl.ds(v * 16, 16)
                ob[j, sl] = y1[j, sl] * a1 + y2[j, sl] * a2 + shb[j, sl]
                return 0

            lax.fori_loop(0, D // 16, body, 0)
        pltpu.sync_copy(ob, out_hbm.at[pl.ds(t0 + c * 16, 16)])


def _combine(y, p1, p2, w1, w2, sh, interpret=False):
    mesh = plsc.VectorSubcoreMesh(core_axis_name="c", subcore_axis_name="s")
    kfn = pl.kernel(
        _combine_body,
        out_type=jax.ShapeDtypeStruct((T, D), jnp.float32),
        mesh=mesh,
        scratch_types=[
            pltpu.VMEM((4, 16), jnp.int32),
            pltpu.VMEM((4, 16), jnp.int32),
            pltpu.VMEM((TPW,), jnp.float32),
            pltpu.VMEM((TPW,), jnp.float32),
            pltpu.VMEM((16, D), jnp.float32),
            pltpu.VMEM((16, D), jnp.float32),
            pltpu.VMEM((16, D), jnp.float32),
            pltpu.VMEM((16, D), jnp.float32),
            pltpu.SemaphoreType.DMA,
            pltpu.SemaphoreType.DMA,
            pltpu.SemaphoreType.DMA,
        ],
        compiler_params=pltpu.CompilerParams(needs_layout_passes=False),
        interpret=interpret,
    )
    return kfn(y, p1, p2, w1, w2, sh)


def kernel(hidden_states, gate_w, e_score_correction_bias,
           w_gate_proj, w_up_proj, w_down_proj,
           ws_gate_proj, ws_up_proj, ws_down_proj):
    x = hidden_states
    bias2d = e_score_correction_bias.reshape(1, E)
    i1, i2, w1, w2, cnt = _router(x, gate_w, bias2d)
    cnt = cnt.reshape(NW, E)
    w1 = w1.reshape(T)
    w2 = w2.reshape(T)
    xg, p1, p2, be = _dispatch(x, i1.reshape(T), i2.reshape(T), cnt)
    y = _grouped_gemm(be, xg, w_gate_proj, w_up_proj, w_down_proj)
    sh = _shared(x, ws_gate_proj, ws_up_proj, ws_down_proj)
    return _combine(y, p1, p2, w1, w2, sh)


# speedup vs baseline: 1.9499x; 1.1458x over previous
import functools

import jax
import jax.numpy as jnp
from jax import lax
from jax.experimental import pallas as pl
from jax.experimental.pallas import tpu as pltpu
from jax.experimental.pallas import tpu_sc as plsc

T = 2048
D = 1024
E = 16
K = 2
F = 512
NG = 4
SH_F = 1024
RSF = 2.5

NC = 2
NS = 16
NW = NC * NS
TPW = T // NW
B = 128
NB = (T * K) // B + E
NPAD = NB * B
RB = 256
NEG = -1e30


def _router_body(x_ref, gw_ref, b_ref, i1_ref, i2_ref, w1_ref, w2_ref,
                 cnt_ref):
    xb = x_ref[...]
    gw = gw_ref[...]
    logits = jnp.dot(xb, gw, preferred_element_type=jnp.float32)
    s = jax.nn.sigmoid(logits)
    sc = s + b_ref[...]

    gs = []
    for g in range(NG):
        a = sc[:, 4 * g + 0:4 * g + 1]
        b2 = sc[:, 4 * g + 1:4 * g + 2]
        c = sc[:, 4 * g + 2:4 * g + 3]
        d = sc[:, 4 * g + 3:4 * g + 4]
        hi1 = jnp.maximum(a, b2)
        lo1 = jnp.minimum(a, b2)
        hi2 = jnp.maximum(c, d)
        lo2 = jnp.minimum(c, d)
        top1 = jnp.maximum(hi1, hi2)
        sec = jnp.maximum(jnp.minimum(hi1, hi2),
                          jnp.where(hi1 >= hi2, lo1, lo2))
        gs.append(top1 + sec)

    bv = gs[0]
    bi = jnp.zeros_like(gs[0], dtype=jnp.int32)
    for g in range(1, NG):
        take = gs[g] > bv
        bv = jnp.where(take, gs[g], bv)
        bi = jnp.where(take, g, bi)
    g1 = bi
    bv = jnp.where(g1 == 0, NEG, gs[0])
    bi = jnp.zeros_like(g1)
    for g in range(1, NG):
        cand = jnp.where(g1 == g, NEG, gs[g])
        take = cand > bv
        bv = jnp.where(take, cand, bv)
        bi = jnp.where(take, g, bi)
    g2 = bi

    rb = RB
    lane = lax.broadcasted_iota(jnp.int32, (rb, E), 1)
    gl = lane // (E // NG)
    m = (gl == g1) | (gl == g2)
    tmp = jnp.where(m, sc, 0.0)

    mx1 = jnp.max(tmp, axis=1, keepdims=True)
    i1 = jnp.min(jnp.where(tmp == mx1, lane, E), axis=1, keepdims=True)
    oh1 = lane == i1
    s1 = jnp.sum(jnp.where(oh1, s, 0.0), axis=1, keepdims=True)
    tmp2 = jnp.where(oh1, NEG, tmp)
    mx2 = jnp.max(tmp2, axis=1, keepdims=True)
    i2 = jnp.min(jnp.where(tmp2 == mx2, lane, E), axis=1, keepdims=True)
    oh2 = lane == i2
    s2 = jnp.sum(jnp.where(oh2, s, 0.0), axis=1, keepdims=True)

    tot = s1 + s2 + 1e-20
    i1_ref[...] = i1
    i2_ref[...] = i2
    w1_ref[...] = s1 / tot * RSF
    w2_ref[...] = s2 / tot * RSF

    oh = oh1.astype(jnp.int32) + oh2.astype(jnp.int32)
    rows = []
    for q in range(RB // TPW):
        sl = slice(q * TPW, (q + 1) * TPW)
        rows.append(jnp.sum(oh[sl, :], axis=0, keepdims=True))
    cnt_ref[...] = jnp.reshape(jnp.concatenate(rows, axis=0), (RB // TPW, 1, E))


def _router(x, gate_w, bias2d, interpret=False):
    out_shapes = (
        jax.ShapeDtypeStruct((T, 1), jnp.int32),
        jax.ShapeDtypeStruct((T, 1), jnp.int32),
        jax.ShapeDtypeStruct((T, 1), jnp.float32),
        jax.ShapeDtypeStruct((T, 1), jnp.float32),
        jax.ShapeDtypeStruct((NW, 1, E), jnp.int32),
    )
    nq = RB // TPW
    return pl.pallas_call(
        _router_body,
        grid=(T // RB,),
        in_specs=[
            pl.BlockSpec((RB, D), lambda i: (i, 0)),
            pl.BlockSpec((D, E), lambda i: (0, 0)),
            pl.BlockSpec((1, E), lambda i: (0, 0)),
        ],
        out_specs=(
            pl.BlockSpec((RB, 1), lambda i: (i, 0)),
            pl.BlockSpec((RB, 1), lambda i: (i, 0)),
            pl.BlockSpec((RB, 1), lambda i: (i, 0)),
            pl.BlockSpec((RB, 1), lambda i: (i, 0)),
            pl.BlockSpec((nq, 1, E), lambda i: (i, 0, 0)),
        ),
        out_shape=out_shapes,
        interpret=interpret,
    )(x, gate_w, bias2d)


def _dispatch_body(x_hbm, i1_hbm, i2_hbm, cnt_hbm,
                   xg_hbm, p1_hbm, p2_hbm, be_hbm,
                   cntbuf, p1buf, p2buf, i1buf, i2buf,
                   xrows, bebuf, sem1, sem2):
    cid = lax.axis_index("c")
    sid = lax.axis_index("s")
    wid = sid * NC + cid
    t0 = wid * TPW

    pltpu.sync_copy(cnt_hbm, cntbuf)
    pltpu.sync_copy(i1_hbm.at[pl.ds(t0, TPW)], i1buf)
    pltpu.sync_copy(i2_hbm.at[pl.ds(t0, TPW)], i2buf)

    tot = jnp.zeros((E,), jnp.int32)
    pre = jnp.zeros((E,), jnp.int32)
    for w in range(NW):
        row = cntbuf[w, :]
        tot = tot + row
        pre = pre + row * (w < wid).astype(jnp.int32)
    padcnt = ((tot + (B - 1)) >> 7) << 7
    csum = jnp.cumsum(padcnt)
    off = csum - padcnt
    base = off + pre

    @pl.when(wid == 0)
    def _():
        nxt = csum >> 7
        for c in range(NB // 16):
            ids = lax.iota(jnp.int32, 16) + c * 16
            acc = jnp.zeros((16,), jnp.int32)
            for e in range(E):
                acc = acc + (ids >= nxt[e]).astype(jnp.int32)
            bebuf[pl.ds(c * 16, 16)] = jnp.minimum(acc, E - 1)
        pltpu.sync_copy(bebuf, be_hbm)

    lane_ids = lax.iota(jnp.int32, 16)
    for c in range(TPW // 16):
        for ibuf, pbuf in ((i1buf, p1buf), (i2buf, p2buf)):
            iv = ibuf[pl.ds(c * 16, 16)]
            pos = jnp.zeros((16,), jnp.int32)
            for e in range(E):
                m = iv == e
                mi = m.astype(jnp.int32)
                cs = jnp.cumsum(mi)
                pos = pos + jnp.where(m, base[e] + cs - 1, 0)
                base = base + jnp.where(lane_ids == e, jnp.sum(mi), 0)
            pbuf[c, :] = pos
        pltpu.sync_copy(x_hbm.at[pl.ds(t0 + c * 16, 16)], xrows)
        cp1 = pltpu.async_copy(xrows, xg_hbm.at[p1buf.at[c]], sem1)
        cp2 = pltpu.async_copy(xrows, xg_hbm.at[p2buf.at[c]], sem2)
        cp1.wait()
        cp2.wait()

    pltpu.sync_copy(p1buf, p1_hbm.at[pl.ds(wid * 4, 4)])
    pltpu.sync_copy(p2buf, p2_hbm.at[pl.ds(wid * 4, 4)])


def _dispatch(x, i1, i2, cnt, interpret=False):
    mesh = plsc.VectorSubcoreMesh(core_axis_name="c", subcore_axis_name="s")
    kfn = pl.kernel(
        _dispatch_body,
        out_type=(
            jax.ShapeDtypeStruct((NPAD, D), jnp.float32),
            jax.ShapeDtypeStruct((T // 16, 16), jnp.int32),
            jax.ShapeDtypeStruct((T // 16, 16), jnp.int32),
            jax.ShapeDtypeStruct((NB,), jnp.int32),
        ),
        mesh=mesh,
        scratch_types=[
            pltpu.VMEM((NW, E), jnp.int32),
            pltpu.VMEM((4, 16), jnp.int32),
            pltpu.VMEM((4, 16), jnp.int32),
            pltpu.VMEM((TPW,), jnp.int32),
            pltpu.VMEM((TPW,), jnp.int32),
            pltpu.VMEM((16, D), jnp.float32),
            pltpu.VMEM((NB,), jnp.int32),
            pltpu.SemaphoreType.DMA,
            pltpu.SemaphoreType.DMA,
        ],
        compiler_params=pltpu.CompilerParams(needs_layout_passes=False),
        interpret=interpret,
    )
    return kfn(x, i1, i2, cnt)


def _gemm_body(be_ref, xg_ref, wg_ref, wu_ref, wd_ref, y_ref):
    xb = xg_ref[...].astype(jnp.bfloat16)
    wg = wg_ref[0].astype(jnp.bfloat16)
    wu = wu_ref[0].astype(jnp.bfloat16)
    h1 = jnp.dot(xb, wg, preferred_element_type=jnp.float32)
    h2 = jnp.dot(xb, wu, preferred_element_type=jnp.float32)
    act = (h1 * jax.nn.sigmoid(h1) * h2).astype(jnp.bfloat16)
    wd = wd_ref[0].astype(jnp.bfloat16)
    y_ref[...] = jnp.dot(act, wd, preferred_element_type=jnp.float32)


def _grouped_gemm(be, xg, wg, wu, wd, interpret=False):
    grid_spec = pltpu.PrefetchScalarGridSpec(
        num_scalar_prefetch=1,
        grid=(NB,),
        in_specs=[
            pl.BlockSpec((B, D), lambda i, be_r: (i, 0)),
            pl.BlockSpec((1, D, F), lambda i, be_r: (be_r[i], 0, 0)),
            pl.BlockSpec((1, D, F), lambda i, be_r: (be_r[i], 0, 0)),
            pl.BlockSpec((1, F, D), lambda i, be_r: (be_r[i], 0, 0)),
        ],
        out_specs=pl.BlockSpec((B, D), lambda i, be_r: (i, 0)),
    )
    return pl.pallas_call(
        _gemm_body,
        grid_spec=grid_spec,
        out_shape=jax.ShapeDtypeStruct((NPAD, D), jnp.float32),
        interpret=interpret,
    )(be, xg, wg, wu, wd)


def _shared_body(x_ref, wg_ref, wu_ref, wd_ref, o_ref):
    xb = x_ref[...].astype(jnp.bfloat16)
    wg = wg_ref[...].astype(jnp.bfloat16)
    wu = wu_ref[...].astype(jnp.bfloat16)
    h1 = jnp.dot(xb, wg, preferred_element_type=jnp.float32)
    h2 = jnp.dot(xb, wu, preferred_element_type=jnp.float32)
    act = (h1 * jax.nn.sigmoid(h1) * h2).astype(jnp.bfloat16)
    wd = wd_ref[...].astype(jnp.bfloat16)
    o_ref[...] = jnp.dot(act, wd, preferred_element_type=jnp.float32)


def _shared(x, wsg, wsu, wsd, interpret=False):
    MB = 256
    return pl.pallas_call(
        _shared_body,
        grid=(T // MB,),
        in_specs=[
            pl.BlockSpec((MB, D), lambda i: (i, 0)),
            pl.BlockSpec((D, SH_F), lambda i: (0, 0)),
            pl.BlockSpec((D, SH_F), lambda i: (0, 0)),
            pl.BlockSpec((SH_F, D), lambda i: (0, 0)),
        ],
        out_specs=pl.BlockSpec((MB, D), lambda i: (i, 0)),
        out_shape=jax.ShapeDtypeStruct((T, D), jnp.float32),
        interpret=interpret,
    )(x, wsg, wsu, wsd)


def _combine_body(y_hbm, p1_hbm, p2_hbm, w1_hbm, w2_hbm, sh_hbm, out_hbm,
                  p1buf, p2buf, w1buf, w2buf,
                  y1a, y2a, sha, y1b, y2b, shb, ob,
                  s1a, s2a, s3a, s1b, s2b, s3b):
    cid = lax.axis_index("c")
    sid = lax.axis_index("s")
    wid = sid * NC + cid
    t0 = wid * TPW

    pltpu.sync_copy(p1_hbm.at[pl.ds(wid * 4, 4)], p1buf)
    pltpu.sync_copy(p2_hbm.at[pl.ds(wid * 4, 4)], p2buf)
    pltpu.sync_copy(w1_hbm.at[pl.ds(t0, TPW)], w1buf)
    pltpu.sync_copy(w2_hbm.at[pl.ds(t0, TPW)], w2buf)

    bufs = ((y1a, y2a, sha, s1a, s2a, s3a), (y1b, y2b, shb, s1b, s2b, s3b))

    def issue(c):
        y1, y2, sh, s1, s2, s3 = bufs[c % 2]
        return (
            pltpu.async_copy(y_hbm.at[p1buf.at[c]], y1, s1),
            pltpu.async_copy(y_hbm.at[p2buf.at[c]], y2, s2),
            pltpu.async_copy(sh_hbm.at[pl.ds(t0 + c * 16, 16)], sh, s3),
        )

    nchunk = TPW // 16
    pending = issue(0)
    for c in range(nchunk):
        for cp in pending:
            cp.wait()
        if c + 1 < nchunk:
            nxt = issue(c + 1)
        y1, y2, sh = bufs[c % 2][:3]
        w1v = w1buf[pl.ds(c * 16, 16)]
        w2v = w2buf[pl.ds(c * 16, 16)]
        for j in range(16):
            a1 = w1v[j]
            a2 = w2v[j]

            def body(v, _, j=j, a1=a1, a2=a2):
                for u in range(4):
                    sl = pl.ds(v * 64 + u * 16, 16)
                    ob[j, sl] = y1[j, sl] * a1 + y2[j, sl] * a2 + sh[j, sl]
                return 0

            lax.fori_loop(0, D // 64, body, 0)
        pltpu.sync_copy(ob, out_hbm.at[pl.ds(t0 + c * 16, 16)])
        if c + 1 < nchunk:
            pending = nxt


def _combine(y, p1, p2, w1, w2, sh, interpret=False):
    mesh = plsc.VectorSubcoreMesh(core_axis_name="c", subcore_axis_name="s")
    kfn = pl.kernel(
        _combine_body,
        out_type=jax.ShapeDtypeStruct((T, D), jnp.float32),
        mesh=mesh,
        scratch_types=[
            pltpu.VMEM((4, 16), jnp.int32),
            pltpu.VMEM((4, 16), jnp.int32),
            pltpu.VMEM((TPW,), jnp.float32),
            pltpu.VMEM((TPW,), jnp.float32),
            pltpu.VMEM((16, D), jnp.float32),
            pltpu.VMEM((16, D), jnp.float32),
            pltpu.VMEM((16, D), jnp.float32),
            pltpu.VMEM((16, D), jnp.float32),
            pltpu.VMEM((16, D), jnp.float32),
            pltpu.VMEM((16, D), jnp.float32),
            pltpu.VMEM((16, D), jnp.float32),
            pltpu.SemaphoreType.DMA,
            pltpu.SemaphoreType.DMA,
            pltpu.SemaphoreType.DMA,
            pltpu.SemaphoreType.DMA,
            pltpu.SemaphoreType.DMA,
            pltpu.SemaphoreType.DMA,
        ],
        compiler_params=pltpu.CompilerParams(needs_layout_passes=False),
        interpret=interpret,
    )
    return kfn(y, p1, p2, w1, w2, sh)


def kernel(hidden_states, gate_w, e_score_correction_bias,
           w_gate_proj, w_up_proj, w_down_proj,
           ws_gate_proj, ws_up_proj, ws_down_proj):
    x = hidden_states
    bias2d = e_score_correction_bias.reshape(1, E)
    i1, i2, w1, w2, cnt = _router(x, gate_w, bias2d)
    cnt = cnt.reshape(NW, E)
    w1 = w1.reshape(T)
    w2 = w2.reshape(T)
    xg, p1, p2, be = _dispatch(x, i1.reshape(T), i2.reshape(T), cnt)
    y = _grouped_gemm(be, xg, w_gate_proj, w_up_proj, w_down_proj)
    sh = _shared(x, ws_gate_proj, ws_up_proj, ws_down_proj)
    return _combine(y, p1, p2, w1, w2, sh)
